# Initial kernel scaffold; baseline (speedup 1.0000x reference)
import jax
import jax.numpy as jnp
from jax.experimental import pallas as pl


def kernel(x, edge_index, edge_attr, params):
    raise NotImplementedError("write your pallas kernel here")


# speedup vs baseline: 1.9405x; 1.9405x over previous
import functools

import jax
import jax.numpy as jnp
from jax import lax
from jax.experimental import pallas as pl
from jax.experimental.pallas import tpu as pltpu
from jax.experimental.pallas import tpu_sc as plsc

N = 10000
E = 160000
F = 16
T = 8
H = 128
NB = 8
OUT = 3

NC = 2
NS = 16
NW = NC * NS
EW = E // NW
K = 40
NCHUNK = EW // K
RW = 632
RW_LAST = N - 15 * RW


def _build_segsum(weighted: bool):
  scratch = [
      pltpu.VMEM_SHARED((N, H), jnp.float32),
      pltpu.VMEM((K, H), jnp.float32),
      pltpu.VMEM((K,), jnp.int32),
      pltpu.VMEM((K,), jnp.int32),
      pltpu.VMEM((K, H), jnp.float32),
      pltpu.SemaphoreType.DMA,
  ]
  if weighted:
    scratch.insert(4, pltpu.VMEM((K, H), jnp.float32))

  mesh = plsc.VectorSubcoreMesh(core_axis_name="c", subcore_axis_name="s")

  @functools.partial(
      pl.kernel, mesh=mesh,
      out_type=[jax.ShapeDtypeStruct((N, H), jnp.float32),
                jax.ShapeDtypeStruct((N, H), jnp.float32)],
      scratch_types=scratch,
  )
  def seg(*refs):
    if weighted:
      (h_hbm, src_hbm, dst_hbm, w_hbm, z_hbm, out0, out1,
       acc, rows, sidx, didx, wrows, cbuf, sem) = refs
    else:
      (h_hbm, src_hbm, dst_hbm, z_hbm, out0, out1,
       acc, rows, sidx, didx, cbuf, sem) = refs
    cid = lax.axis_index("c")
    sid = lax.axis_index("s")
    wid = cid * NS + sid

    def _per_slice(fn):
      @pl.when(sid < NS - 1)
      def _():
        fn(sid * RW, RW)

      @pl.when(sid == NS - 1)
      def _():
        fn((NS - 1) * RW, RW_LAST)

    _per_slice(lambda o, s: pltpu.sync_copy(z_hbm.at[pl.ds(o, s)],
                                            acc.at[pl.ds(o, s)]))
    plsc.subcore_barrier()

    def body(c, carry):
      base = wid * EW + c * K
      pltpu.sync_copy(src_hbm.at[pl.ds(base, K)], sidx)
      pltpu.sync_copy(dst_hbm.at[pl.ds(base, K)], didx)
      pltpu.async_copy(h_hbm.at[sidx], rows, sem).wait()
      if weighted:
        pltpu.sync_copy(w_hbm.at[pl.ds(base, K)], wrows)
        for i in range(K):
          for j in range(H // 16):
            s = pl.ds(j * 16, 16)
            rows[i, s] = rows[i, s] * wrows[i, s]
      pltpu.sync_copy(rows, acc.at[didx], add=True)
      return carry

    lax.fori_loop(0, NCHUNK, body, 0)
    plsc.subcore_barrier()

    def _chunk_out(oo, sz):
      pltpu.sync_copy(acc.at[pl.ds(oo, sz)], cbuf.at[pl.ds(0, sz)])

      @pl.when(cid == 0)
      def _():
        pltpu.sync_copy(cbuf.at[pl.ds(0, sz)], out0.at[pl.ds(oo, sz)])

      @pl.when(cid == 1)
      def _():
        pltpu.sync_copy(cbuf.at[pl.ds(0, sz)], out1.at[pl.ds(oo, sz)])

    def _copyout(o, s):
      def cbody(kk, carry):
        _chunk_out(o + kk * K, K)
        return carry

      lax.fori_loop(0, s // K, cbody, 0)
      if s % K:
        _chunk_out(o + (s // K) * K, s % K)

    _per_slice(_copyout)

  return seg


_segsum_plain = _build_segsum(False)
_segsum_weighted = _build_segsum(True)


def _silu(x):
  return x * jax.nn.sigmoid(x)


BG = 2000


def _gate_body(ea, w1, b1, w2, b2, out):
  ef = _silu(jnp.dot(ea[...], w1[...], preferred_element_type=jnp.float32) + b1[...])
  out[...] = jax.nn.sigmoid(jnp.dot(ef, w2[...], preferred_element_type=jnp.float32) + b2[...])


def _tc_gate(ea_pad, w1, b1, w2, b2):
  return pl.pallas_call(
      _gate_body,
      grid=(E // BG,),
      in_specs=[
          pl.BlockSpec((BG, 8), lambda i: (i, 0)),
          pl.BlockSpec((8, 32), lambda i: (0, 0)),
          pl.BlockSpec((1, 32), lambda i: (0, 0)),
          pl.BlockSpec((32, 1), lambda i: (0, 0)),
          pl.BlockSpec((1, 1), lambda i: (0, 0)),
      ],
      out_specs=pl.BlockSpec((BG, 1), lambda i: (i, 0)),
      out_shape=jax.ShapeDtypeStruct((E, 1), jnp.float32),
  )(ea_pad, w1, b1, w2, b2)


BE = 2000


def _enc_body(xr, x2, gt, ws, wb, b, out):
  phi = jnp.exp(-((xr[...] - gt[...]) ** 2))
  out[...] = (jnp.dot(phi, ws[...], preferred_element_type=jnp.float32)
              + jnp.dot(_silu(x2[...]), wb[...], preferred_element_type=jnp.float32)
              + b[...])


def _tc_enc(xr, x2, gt, ws, wb, b):
  TN = T * N
  return pl.pallas_call(
      _enc_body,
      grid=(TN // BE,),
      in_specs=[
          pl.BlockSpec((BE, F * NB), lambda i: (i, 0)),
          pl.BlockSpec((BE, F), lambda i: (i, 0)),
          pl.BlockSpec((1, F * NB), lambda i: (0, 0)),
          pl.BlockSpec((F * NB, H), lambda i: (0, 0)),
          pl.BlockSpec((F, H), lambda i: (0, 0)),
          pl.BlockSpec((1, H), lambda i: (0, 0)),
      ],
      out_specs=pl.BlockSpec((BE, H), lambda i: (i, 0)),
      out_shape=jax.ShapeDtypeStruct((TN, H), jnp.float32),
  )(xr, x2, gt, ws, wb, b)


BC = 2000


def _comb_body(a, b, gv, hop, hnx):
  s = (a[...] + b[...]) * gv[...]
  hop[...] = s
  hnx[...] = s * gv[...]


def _tc_combine(a, b, gv):
  return pl.pallas_call(
      _comb_body,
      grid=(N // BC,),
      in_specs=[
          pl.BlockSpec((BC, H), lambda i: (i, 0)),
          pl.BlockSpec((BC, H), lambda i: (i, 0)),
          pl.BlockSpec((BC, 1), lambda i: (i, 0)),
      ],
      out_specs=[pl.BlockSpec((BC, H), lambda i: (i, 0)),
                 pl.BlockSpec((BC, H), lambda i: (i, 0))],
      out_shape=[jax.ShapeDtypeStruct((N, H), jnp.float32),
                 jax.ShapeDtypeStruct((N, H), jnp.float32)],
  )(a, b, gv)


BD = 1000
DT = 1.0 / T


def _dense_body(h, hop1, hop2, p3a, p3b, pga, pgb, u, env8, gv,
                aa, rs, gt2, wsk, wbk, bk, wt, bt,
                wch, wce, wca, bc, wth, wte, wta, btau,
                lng, lnb, wsd, wbd, bd,
                hout, hsout, yout):
  hv = h[...]
  g = gv[...]
  hop3 = (p3a[...] + p3b[...]) * g
  agg = pga[...] + pgb[...]
  h1 = hop1[...]
  h2 = hop2[...]
  av = aa[...]
  e0 = jnp.dot(hv, av, preferred_element_type=jnp.float32)
  e1 = jnp.dot(h1, av, preferred_element_type=jnp.float32)
  e2 = jnp.dot(h2, av, preferred_element_type=jnp.float32)
  e3 = jnp.dot(hop3, av, preferred_element_type=jnp.float32)
  m = jnp.maximum(jnp.maximum(e0, e1), jnp.maximum(e2, e3))
  w0 = jnp.exp(e0 - m)
  w1 = jnp.exp(e1 - m)
  w2 = jnp.exp(e2 - m)
  w3 = jnp.exp(e3 - m)
  sw = w0 + w1 + w2 + w3
  comb = (w0 * hv + w1 * h1 + w2 * h2 + w3 * hop3) / sw

  crep = jnp.dot(comb, rs[...], preferred_element_type=jnp.float32)
  phi = jnp.exp(-((crep - gt2[...]) ** 2))
  hk = (jnp.dot(phi, wsk[...], preferred_element_type=jnp.float32)
        + jnp.dot(_silu(comb), wbk[...], preferred_element_type=jnp.float32)
        + bk[...])
  press = jnp.dot(agg, wt[...], preferred_element_type=jnp.float32) + bt[...]
  ha = hk + press

  ev = env8[...]
  zc = (jnp.dot(hv, wch[...], preferred_element_type=jnp.float32)
        + jnp.dot(ev, wce[...], preferred_element_type=jnp.float32)
        + jnp.dot(ha, wca[...], preferred_element_type=jnp.float32) + bc[...])
  pre = jnp.tanh(zc)
  zt = (jnp.dot(hv, wth[...], preferred_element_type=jnp.float32)
        + jnp.dot(ev, wte[...], preferred_element_type=jnp.float32)
        + jnp.dot(ha, wta[...], preferred_element_type=jnp.float32) + btau[...])
  tau = 1.0 + 9.0 * jax.nn.sigmoid(zt)
  hn = hv + DT * (pre - hv) / tau
  mu = jnp.mean(hn, axis=1, keepdims=True)
  va = jnp.mean((hn - mu) ** 2, axis=1, keepdims=True)
  hn = (hn - mu) * jax.lax.rsqrt(va + 1e-5) * lng[...] + lnb[...]
  hnew = hn + u[...]

  hrep = jnp.dot(hnew, rs[...], preferred_element_type=jnp.float32)
  phid = jnp.exp(-((hrep - gt2[...]) ** 2))
  y = jax.nn.softplus(jnp.dot(phid, wsd[...], preferred_element_type=jnp.float32)
                      + jnp.dot(_silu(hnew), wbd[...], preferred_element_type=jnp.float32)
                      + bd[...])
  hout[...] = hnew
  hsout[...] = hnew * g
  yout[...] = y


def _tc_dense(args):
  blk = lambda shp: pl.BlockSpec(shp, lambda i: (i, 0))
  full = lambda shp: pl.BlockSpec(shp, lambda i: (0, 0))
  in_specs = (
      [blk((BD, H))] * 8
      + [blk((BD, 8)), blk((BD, 1))]
      + [full((H, 1)), full((H, H * NB)), full((1, H * NB)),
         full((H * NB, H)), full((H, H)), full((1, H)),
         full((H, H)), full((1, H)),
         full((H, H)), full((8, H)), full((H, H)), full((1, H)),
         full((H, H)), full((8, H)), full((H, H)), full((1, H)),
         full((1, H)), full((1, H)),
         full((H * NB, H)), full((H, H)), full((1, H))]
  )
  return pl.pallas_call(
      _dense_body,
      grid=(N // BD,),
      in_specs=in_specs,
      out_specs=[blk((BD, H)), blk((BD, H)), blk((BD, H))],
      out_shape=[jax.ShapeDtypeStruct((N, H), jnp.float32),
                 jax.ShapeDtypeStruct((N, H), jnp.float32),
                 jax.ShapeDtypeStruct((N, H), jnp.float32)],
  )(*args)


def _impl(x, edge_index, edge_attr, params):
  src = edge_index[0]
  dst = edge_index[1]
  zNH = jnp.zeros((N, H), jnp.float32)

  ea_pad = jnp.pad(edge_attr, ((0, 0), (0, 4)))
  lp = params['larval']
  w1p = jnp.pad(lp['W1'], ((0, 4), (0, 0)))
  gate = _tc_gate(ea_pad, w1p, lp['b1'].reshape(1, 32), lp['w2'],
                  lp['b2'].reshape(1, 1))[:, 0]

  d0, d1 = _segsum_plain(jnp.ones((N, H), jnp.float32), src, dst, zNH)
  deg = d0[:, 0] + d1[:, 0]
  g = jax.lax.rsqrt(jnp.clip(deg, 1.0, None))
  gv = g[:, None]

  enc = params['enc']
  invd = enc['invd']
  x2 = x.reshape(T * N, F)
  xr = jnp.repeat(x2, NB, axis=1) * invd
  gtE = (jnp.tile(enc['grid'], F) * invd).reshape(1, F * NB)
  u_all = _tc_enc(xr, x2, gtE, enc['Ws'], enc['Wb'],
                  enc['b'].reshape(1, H)).reshape(T, N, H)

  env8 = jnp.pad(x[:, :, 8:13], ((0, 0), (0, 0), (0, 3)))

  kh = params['khop']
  kk = kh['kan']
  rep = jnp.zeros((H, H * NB), jnp.float32)
  rep = rep.at[jnp.repeat(jnp.arange(H), NB), jnp.arange(H * NB)].set(1.0)
  invdK = kk['invd']
  rs = rep * invdK
  gt2 = (jnp.tile(kk['grid'], H) * invdK).reshape(1, H * NB)
  cell = params['cell']
  Wc, Wtau = cell['Wc'], cell['Wtau']
  dec = params['dec']
  wsd = jnp.pad(dec['Ws'], ((0, 0), (0, H - OUT)))
  wbd = jnp.pad(dec['Wb'], ((0, 0), (0, H - OUT)))
  bd = jnp.pad(dec['b'], (0, H - OUT)).reshape(1, H)
  dense_w = [
      kh['att_a'].reshape(H, 1), rs, gt2, kk['Ws'], kk['Wb'],
      kk['b'].reshape(1, H), lp['Wt'], lp['bt'].reshape(1, H),
      Wc[:H], jnp.pad(Wc[H:H + 5], ((0, 3), (0, 0))), Wc[H + 5:],
      cell['bc'].reshape(1, H),
      Wtau[:H], jnp.pad(Wtau[H:H + 5], ((0, 3), (0, 0))), Wtau[H + 5:],
      cell['btau'].reshape(1, H),
      params['ln']['g'].reshape(1, H), params['ln']['b'].reshape(1, H),
      wsd, wbd, bd,
  ]

  gate_rows = jnp.broadcast_to(gate[:, None], (E, H))

  h = jnp.broadcast_to(params['h0'].reshape(1, H), (N, H))
  hs = h * gv
  ys = []
  for t in range(T):
    q1a, q1b = _segsum_plain(hs, src, dst, zNH)
    hop1, hs1 = _tc_combine(q1a, q1b, gv)
    q2a, q2b = _segsum_plain(hs1, src, dst, zNH)
    hop2, hs2 = _tc_combine(q2a, q2b, gv)
    p3a, p3b = _segsum_plain(hs2, src, dst, zNH)
    pga, pgb = _segsum_weighted(h, src, dst, gate_rows, zNH)
    h, hs, y = _tc_dense([h, hop1, hop2, p3a, p3b, pga, pgb,
                          u_all[t], env8[t], gv] + dense_w)
    ys.append(y[:, :OUT])
  return jnp.stack(ys, axis=0)


def kernel(x, edge_index, edge_attr, params):
  return _impl(x, edge_index, edge_attr, params)


# speedup vs baseline: 2.6040x; 1.3420x over previous
import functools

import jax
import jax.numpy as jnp
from jax import lax
from jax.experimental import pallas as pl
from jax.experimental.pallas import tpu as pltpu
from jax.experimental.pallas import tpu_sc as plsc

N = 10000
E = 160000
F = 16
T = 8
H = 128
NB = 8
OUT = 3

NC = 2
NS = 16
NW = NC * NS
EW = E // NW
K = 40
NCHUNK = EW // K
RW = 632
RW_LAST = N - 15 * RW


def _build_segsum(weighted: bool):
  scratch = [
      pltpu.VMEM_SHARED((N, H), jnp.float32),
      pltpu.VMEM((K, H), jnp.float32),
      pltpu.VMEM((K, H), jnp.float32),
      pltpu.VMEM((K,), jnp.int32),
      pltpu.VMEM((K,), jnp.int32),
      pltpu.VMEM((K,), jnp.int32),
      pltpu.VMEM((K,), jnp.int32),
      pltpu.VMEM((K, H), jnp.float32),
      pltpu.SemaphoreType.DMA,
      pltpu.SemaphoreType.DMA,
  ]
  if weighted:
    scratch[7:7] = [pltpu.VMEM((K, H), jnp.float32),
                    pltpu.VMEM((K, H), jnp.float32)]

  mesh = plsc.VectorSubcoreMesh(core_axis_name="c", subcore_axis_name="s")

  @functools.partial(
      pl.kernel, mesh=mesh,
      out_type=[jax.ShapeDtypeStruct((N, H), jnp.float32),
                jax.ShapeDtypeStruct((N, H), jnp.float32)],
      scratch_types=scratch,
  )
  def seg(*refs):
    if weighted:
      (h_hbm, src_hbm, dst_hbm, w_hbm, z_hbm, out0, out1,
       acc, rows0, rows1, sidx0, sidx1, didx0, didx1,
       wrows0, wrows1, cbuf, sem0, sem1) = refs
      wrows = (wrows0, wrows1)
    else:
      (h_hbm, src_hbm, dst_hbm, z_hbm, out0, out1,
       acc, rows0, rows1, sidx0, sidx1, didx0, didx1,
       cbuf, sem0, sem1) = refs
      wrows = (None, None)
    rows = (rows0, rows1)
    sidx = (sidx0, sidx1)
    didx = (didx0, didx1)
    sem = (sem0, sem1)
    cid = lax.axis_index("c")
    sid = lax.axis_index("s")
    wid = cid * NS + sid
    wstart = wid * EW

    def _per_slice(fn):
      @pl.when(sid < NS - 1)
      def _():
        fn(sid * RW, RW)

      @pl.when(sid == NS - 1)
      def _():
        fn((NS - 1) * RW, RW_LAST)

    def _prefetch(b, c):
      base = wstart + c * K
      pltpu.sync_copy(src_hbm.at[pl.ds(base, K)], sidx[b])
      pltpu.sync_copy(dst_hbm.at[pl.ds(base, K)], didx[b])
      pltpu.async_copy(h_hbm.at[sidx[b]], rows[b], sem[b])

    def _process(b, c):
      pltpu.make_async_copy(h_hbm.at[sidx[b]], rows[b], sem[b]).wait()
      if weighted:
        base = wstart + c * K
        pltpu.sync_copy(w_hbm.at[pl.ds(base, K)], wrows[b])
        for i in range(K):
          for j in range(H // 16):
            s = pl.ds(j * 16, 16)
            rows[b][i, s] = rows[b][i, s] * wrows[b][i, s]
      pltpu.sync_copy(rows[b], acc.at[didx[b]], add=True)

    _per_slice(lambda o, s: pltpu.sync_copy(z_hbm.at[pl.ds(o, s)],
                                            acc.at[pl.ds(o, s)]))
    _prefetch(0, 0)
    _prefetch(1, 1)
    plsc.subcore_barrier()

    NPAIR = (NCHUNK - 1) // 2

    def body(j, carry):
      c0 = 2 * j
      _process(0, c0)
      _prefetch(0, c0 + 2)
      _process(1, c0 + 1)

      @pl.when(j < NPAIR - 1)
      def _():
        _prefetch(1, c0 + 3)

      return carry

    lax.fori_loop(0, NPAIR, body, 0)
    _process(0, NCHUNK - 1)
    plsc.subcore_barrier()

    def _chunk_out(oo, sz):
      pltpu.sync_copy(acc.at[pl.ds(oo, sz)], cbuf.at[pl.ds(0, sz)])

      @pl.when(cid == 0)
      def _():
        pltpu.sync_copy(cbuf.at[pl.ds(0, sz)], out0.at[pl.ds(oo, sz)])

      @pl.when(cid == 1)
      def _():
        pltpu.sync_copy(cbuf.at[pl.ds(0, sz)], out1.at[pl.ds(oo, sz)])

    def _copyout(o, s):
      def cbody(kk, carry):
        _chunk_out(o + kk * K, K)
        return carry

      lax.fori_loop(0, s // K, cbody, 0)
      if s % K:
        _chunk_out(o + (s // K) * K, s % K)

    _per_slice(_copyout)

  return seg


_segsum_plain = _build_segsum(False)
_segsum_weighted = _build_segsum(True)


def _silu(x):
  return x * jax.nn.sigmoid(x)


BG = 2000


def _gate_body(ea, w1, b1, w2, b2, out):
  ef = _silu(jnp.dot(ea[...], w1[...], preferred_element_type=jnp.float32) + b1[...])
  out[...] = jax.nn.sigmoid(jnp.dot(ef, w2[...], preferred_element_type=jnp.float32) + b2[...])


def _tc_gate(ea_pad, w1, b1, w2, b2):
  return pl.pallas_call(
      _gate_body,
      grid=(E // BG,),
      in_specs=[
          pl.BlockSpec((BG, 8), lambda i: (i, 0)),
          pl.BlockSpec((8, 32), lambda i: (0, 0)),
          pl.BlockSpec((1, 32), lambda i: (0, 0)),
          pl.BlockSpec((32, 1), lambda i: (0, 0)),
          pl.BlockSpec((1, 1), lambda i: (0, 0)),
      ],
      out_specs=pl.BlockSpec((BG, 1), lambda i: (i, 0)),
      out_shape=jax.ShapeDtypeStruct((E, 1), jnp.float32),
  )(ea_pad, w1, b1, w2, b2)


BE = 2000


def _enc_body(xr, x2, gt, ws, wb, b, out):
  phi = jnp.exp(-((xr[...] - gt[...]) ** 2))
  out[...] = (jnp.dot(phi, ws[...], preferred_element_type=jnp.float32)
              + jnp.dot(_silu(x2[...]), wb[...], preferred_element_type=jnp.float32)
              + b[...])


def _tc_enc(xr, x2, gt, ws, wb, b):
  TN = T * N
  return pl.pallas_call(
      _enc_body,
      grid=(TN // BE,),
      in_specs=[
          pl.BlockSpec((BE, F * NB), lambda i: (i, 0)),
          pl.BlockSpec((BE, F), lambda i: (i, 0)),
          pl.BlockSpec((1, F * NB), lambda i: (0, 0)),
          pl.BlockSpec((F * NB, H), lambda i: (0, 0)),
          pl.BlockSpec((F, H), lambda i: (0, 0)),
          pl.BlockSpec((1, H), lambda i: (0, 0)),
      ],
      out_specs=pl.BlockSpec((BE, H), lambda i: (i, 0)),
      out_shape=jax.ShapeDtypeStruct((TN, H), jnp.float32),
  )(xr, x2, gt, ws, wb, b)


BC = 2000


def _comb_body(a, b, gv, hop, hnx):
  s = (a[...] + b[...]) * gv[...]
  hop[...] = s
  hnx[...] = s * gv[...]


def _tc_combine(a, b, gv):
  return pl.pallas_call(
      _comb_body,
      grid=(N // BC,),
      in_specs=[
          pl.BlockSpec((BC, H), lambda i: (i, 0)),
          pl.BlockSpec((BC, H), lambda i: (i, 0)),
          pl.BlockSpec((BC, 1), lambda i: (i, 0)),
      ],
      out_specs=[pl.BlockSpec((BC, H), lambda i: (i, 0)),
                 pl.BlockSpec((BC, H), lambda i: (i, 0))],
      out_shape=[jax.ShapeDtypeStruct((N, H), jnp.float32),
                 jax.ShapeDtypeStruct((N, H), jnp.float32)],
  )(a, b, gv)


BD = 1000
DT = 1.0 / T


def _dense_body(h, hop1, hop2, p3a, p3b, pga, pgb, u, env8, gv,
                aa, rs, gt2, wsk, wbk, bk, wt, bt,
                wch, wce, wca, bc, wth, wte, wta, btau,
                lng, lnb, wsd, wbd, bd,
                hout, hsout, yout):
  hv = h[...]
  g = gv[...]
  hop3 = (p3a[...] + p3b[...]) * g
  agg = pga[...] + pgb[...]
  h1 = hop1[...]
  h2 = hop2[...]
  av = aa[...]
  e0 = jnp.dot(hv, av, preferred_element_type=jnp.float32)
  e1 = jnp.dot(h1, av, preferred_element_type=jnp.float32)
  e2 = jnp.dot(h2, av, preferred_element_type=jnp.float32)
  e3 = jnp.dot(hop3, av, preferred_element_type=jnp.float32)
  m = jnp.maximum(jnp.maximum(e0, e1), jnp.maximum(e2, e3))
  w0 = jnp.exp(e0 - m)
  w1 = jnp.exp(e1 - m)
  w2 = jnp.exp(e2 - m)
  w3 = jnp.exp(e3 - m)
  sw = w0 + w1 + w2 + w3
  comb = (w0 * hv + w1 * h1 + w2 * h2 + w3 * hop3) / sw

  crep = jnp.dot(comb, rs[...], preferred_element_type=jnp.float32)
  phi = jnp.exp(-((crep - gt2[...]) ** 2))
  hk = (jnp.dot(phi, wsk[...], preferred_element_type=jnp.float32)
        + jnp.dot(_silu(comb), wbk[...], preferred_element_type=jnp.float32)
        + bk[...])
  press = jnp.dot(agg, wt[...], preferred_element_type=jnp.float32) + bt[...]
  ha = hk + press

  ev = env8[...]
  zc = (jnp.dot(hv, wch[...], preferred_element_type=jnp.float32)
        + jnp.dot(ev, wce[...], preferred_element_type=jnp.float32)
        + jnp.dot(ha, wca[...], preferred_element_type=jnp.float32) + bc[...])
  pre = jnp.tanh(zc)
  zt = (jnp.dot(hv, wth[...], preferred_element_type=jnp.float32)
        + jnp.dot(ev, wte[...], preferred_element_type=jnp.float32)
        + jnp.dot(ha, wta[...], preferred_element_type=jnp.float32) + btau[...])
  tau = 1.0 + 9.0 * jax.nn.sigmoid(zt)
  hn = hv + DT * (pre - hv) / tau
  mu = jnp.mean(hn, axis=1, keepdims=True)
  va = jnp.mean((hn - mu) ** 2, axis=1, keepdims=True)
  hn = (hn - mu) * jax.lax.rsqrt(va + 1e-5) * lng[...] + lnb[...]
  hnew = hn + u[...]

  hrep = jnp.dot(hnew, rs[...], preferred_element_type=jnp.float32)
  phid = jnp.exp(-((hrep - gt2[...]) ** 2))
  y = jax.nn.softplus(jnp.dot(phid, wsd[...], preferred_element_type=jnp.float32)
                      + jnp.dot(_silu(hnew), wbd[...], preferred_element_type=jnp.float32)
                      + bd[...])
  hout[...] = hnew
  hsout[...] = hnew * g
  yout[...] = y


def _tc_dense(args):
  blk = lambda shp: pl.BlockSpec(shp, lambda i: (i, 0))
  full = lambda shp: pl.BlockSpec(shp, lambda i: (0, 0))
  in_specs = (
      [blk((BD, H))] * 8
      + [blk((BD, 8)), blk((BD, 1))]
      + [full((H, 1)), full((H, H * NB)), full((1, H * NB)),
         full((H * NB, H)), full((H, H)), full((1, H)),
         full((H, H)), full((1, H)),
         full((H, H)), full((8, H)), full((H, H)), full((1, H)),
         full((H, H)), full((8, H)), full((H, H)), full((1, H)),
         full((1, H)), full((1, H)),
         full((H * NB, H)), full((H, H)), full((1, H))]
  )
  return pl.pallas_call(
      _dense_body,
      grid=(N // BD,),
      in_specs=in_specs,
      out_specs=[blk((BD, H)), blk((BD, H)), blk((BD, H))],
      out_shape=[jax.ShapeDtypeStruct((N, H), jnp.float32),
                 jax.ShapeDtypeStruct((N, H), jnp.float32),
                 jax.ShapeDtypeStruct((N, H), jnp.float32)],
  )(*args)


def _impl(x, edge_index, edge_attr, params):
  src = edge_index[0]
  dst = edge_index[1]
  zNH = jnp.zeros((N, H), jnp.float32)

  ea_pad = jnp.pad(edge_attr, ((0, 0), (0, 4)))
  lp = params['larval']
  w1p = jnp.pad(lp['W1'], ((0, 4), (0, 0)))
  gate = _tc_gate(ea_pad, w1p, lp['b1'].reshape(1, 32), lp['w2'],
                  lp['b2'].reshape(1, 1))[:, 0]

  d0, d1 = _segsum_plain(jnp.ones((N, H), jnp.float32), src, dst, zNH)
  deg = d0[:, 0] + d1[:, 0]
  g = jax.lax.rsqrt(jnp.clip(deg, 1.0, None))
  gv = g[:, None]

  enc = params['enc']
  invd = enc['invd']
  x2 = x.reshape(T * N, F)
  xr = jnp.repeat(x2, NB, axis=1) * invd
  gtE = (jnp.tile(enc['grid'], F) * invd).reshape(1, F * NB)
  u_all = _tc_enc(xr, x2, gtE, enc['Ws'], enc['Wb'],
                  enc['b'].reshape(1, H)).reshape(T, N, H)

  env8 = jnp.pad(x[:, :, 8:13], ((0, 0), (0, 0), (0, 3)))

  kh = params['khop']
  kk = kh['kan']
  rep = jnp.zeros((H, H * NB), jnp.float32)
  rep = rep.at[jnp.repeat(jnp.arange(H), NB), jnp.arange(H * NB)].set(1.0)
  invdK = kk['invd']
  rs = rep * invdK
  gt2 = (jnp.tile(kk['grid'], H) * invdK).reshape(1, H * NB)
  cell = params['cell']
  Wc, Wtau = cell['Wc'], cell['Wtau']
  dec = params['dec']
  wsd = jnp.pad(dec['Ws'], ((0, 0), (0, H - OUT)))
  wbd = jnp.pad(dec['Wb'], ((0, 0), (0, H - OUT)))
  bd = jnp.pad(dec['b'], (0, H - OUT)).reshape(1, H)
  dense_w = [
      kh['att_a'].reshape(H, 1), rs, gt2, kk['Ws'], kk['Wb'],
      kk['b'].reshape(1, H), lp['Wt'], lp['bt'].reshape(1, H),
      Wc[:H], jnp.pad(Wc[H:H + 5], ((0, 3), (0, 0))), Wc[H + 5:],
      cell['bc'].reshape(1, H),
      Wtau[:H], jnp.pad(Wtau[H:H + 5], ((0, 3), (0, 0))), Wtau[H + 5:],
      cell['btau'].reshape(1, H),
      params['ln']['g'].reshape(1, H), params['ln']['b'].reshape(1, H),
      wsd, wbd, bd,
  ]

  gate_rows = jnp.broadcast_to(gate[:, None], (E, H))

  h = jnp.broadcast_to(params['h0'].reshape(1, H), (N, H))
  hs = h * gv
  ys = []
  for t in range(T):
    q1a, q1b = _segsum_plain(hs, src, dst, zNH)
    hop1, hs1 = _tc_combine(q1a, q1b, gv)
    q2a, q2b = _segsum_plain(hs1, src, dst, zNH)
    hop2, hs2 = _tc_combine(q2a, q2b, gv)
    p3a, p3b = _segsum_plain(hs2, src, dst, zNH)
    pga, pgb = _segsum_weighted(h, src, dst, gate_rows, zNH)
    h, hs, y = _tc_dense([h, hop1, hop2, p3a, p3b, pga, pgb,
                          u_all[t], env8[t], gv] + dense_w)
    ys.append(y[:, :OUT])
  return jnp.stack(ys, axis=0)


def kernel(x, edge_index, edge_attr, params):
  return _impl(x, edge_index, edge_attr, params)
